# Initial kernel scaffold; baseline (speedup 1.0000x reference)
import jax
import jax.numpy as jnp
from jax.experimental import pallas as pl


def kernel(x, edge_index_ast, edge_index_cfg, edge_index_pdg, batch, params):
    raise NotImplementedError("write your pallas kernel here")


# speedup vs baseline: 69.3803x; 69.3803x over previous
import functools

import jax
import jax.numpy as jnp
from jax import lax
from jax.experimental import pallas as pl
from jax.experimental.pallas import tpu as pltpu
from jax.experimental.pallas import tpu_sc as plsc

_N = 10000
_E = 320000
_G = 64
_PK = 128
_ADW = 16
_CH = 80
_NTILES = 32
_EPT = _E // _NTILES
_NCH = _EPT // _CH
_BLK = 1000
_ZCH = (_N // _CH + _NTILES // 2 - 1) // (_NTILES // 2)


def _lane_broadcast(v, idx):
    dn = lax.GatherDimensionNumbers(
        offset_dims=(), collapsed_slice_dims=(0,), start_index_map=(0,))
    return lax.gather(v, idx[:, None], dn, slice_sizes=(1,),
                      mode=lax.GatherScatterMode.PROMISE_IN_BOUNDS)


def _sc_edge_body(hx, as0, as1, ad0, ad1, src, dst, out, acc, srcv, dstv,
                  rows, asv0, asv1, adv0, adv1, sem):
    c = lax.axis_index("c")
    s = lax.axis_index("s")
    wid = c * 16 + s
    nrowch = _N // _CH

    pltpu.sync_copy(as0, asv0)
    pltpu.sync_copy(as1, asv1)
    pltpu.sync_copy(ad0, adv0)
    pltpu.sync_copy(ad1, adv1)

    zv = jnp.zeros((16,), jnp.float32)

    def _zbuf(i, carry):
        r = i // (_PK // 16)
        col = (i % (_PK // 16)) * 16
        rows[r, pl.ds(col, 16)] = zv
        return carry

    lax.fori_loop(0, _CH * (_PK // 16), _zbuf, 0)

    for k in range(_ZCH):
        @pl.when(s + 16 * k < nrowch)
        def _():
            pltpu.sync_copy(rows, acc.at[pl.ds((s + 16 * k) * _CH, _CH)])

    plsc.subcore_barrier()

    lanes = lax.iota(jnp.int32, 16)
    lane_consts = [jnp.full((16,), j, jnp.int32) for j in range(16)]

    def _chunk(i, carry):
        base = wid * _EPT + i * _CH
        pltpu.sync_copy(src.at[pl.ds(base, _CH)], srcv)
        pltpu.sync_copy(dst.at[pl.ds(base, _CH)], dstv)
        pltpu.async_copy(hx.at[srcv], rows, sem).wait()

        def _grp(g, carry2):
            off = g * 16
            svec = srcv[pl.ds(off, 16)]
            dvec = dstv[pl.ds(off, 16)]
            al0 = plsc.load_gather(asv0, [svec]) + plsc.load_gather(adv0, [dvec])
            al1 = plsc.load_gather(asv1, [svec]) + plsc.load_gather(adv1, [dvec])
            al0 = jnp.maximum(al0, 0.2 * al0)
            al1 = jnp.maximum(al1, 0.2 * al1)
            ex0v = jnp.exp(al0)
            ex1v = jnp.exp(al1)
            for j in range(16):
                b0 = _lane_broadcast(ex0v, lane_consts[j])
                b1 = _lane_broadcast(ex1v, lane_consts[j])
                exsel = jnp.where(lanes == 0, b0,
                                  jnp.where(lanes == 1, b1, 0.0))
                er = off + j
                rows[er, pl.ds(64, 16)] = exsel
                rows[er, pl.ds(0, 16)] = rows[er, pl.ds(0, 16)] * b0
                rows[er, pl.ds(16, 16)] = rows[er, pl.ds(16, 16)] * b0
                rows[er, pl.ds(32, 16)] = rows[er, pl.ds(32, 16)] * b1
                rows[er, pl.ds(48, 16)] = rows[er, pl.ds(48, 16)] * b1
            return carry2

        lax.fori_loop(0, _CH // 16, _grp, 0)
        pltpu.sync_copy(rows, acc.at[dstv], add=True)
        return carry

    lax.fori_loop(0, _NCH, _chunk, 0)
    plsc.subcore_barrier()

    for k in range(_ZCH):
        @pl.when(s + 16 * k < nrowch)
        def _():
            off = (s + 16 * k) * _CH
            pltpu.sync_copy(acc.at[pl.ds(off, _CH)], out.at[c, pl.ds(off, _CH)])


_sc_edge = functools.partial(
    pl.kernel,
    mesh=plsc.VectorSubcoreMesh(core_axis_name="c", subcore_axis_name="s"),
    compiler_params=pltpu.CompilerParams(needs_layout_passes=False),
    out_type=jax.ShapeDtypeStruct((2, _N, _PK), jnp.float32),
    scratch_types=[
        pltpu.VMEM_SHARED((_N, _PK), jnp.float32),
        pltpu.VMEM((_CH,), jnp.int32),
        pltpu.VMEM((_CH,), jnp.int32),
        pltpu.VMEM((_CH, _PK), jnp.float32),
        pltpu.VMEM((_N,), jnp.float32),
        pltpu.VMEM((_N,), jnp.float32),
        pltpu.VMEM((_N,), jnp.float32),
        pltpu.VMEM((_N,), jnp.float32),
        pltpu.SemaphoreType.DMA,
    ],
)(_sc_edge_body)


def _pack_outputs(h, asd, hx_ref, ad_ref, sf_ref):
    b = h.shape[0]
    a_src = asd[:, 0:2]
    a_dst = asd[:, 2:4]
    al = a_src + a_dst
    ex = jnp.exp(jnp.maximum(al, 0.2 * al))
    zp = jnp.zeros((b, _PK - 66), jnp.float32)
    hx_ref[...] = jnp.concatenate([h, a_src, zp], axis=1)
    ad_ref[...] = jnp.concatenate(
        [a_dst, a_src, jnp.zeros((b, _ADW - 4), jnp.float32)], axis=1)
    sf_ref[...] = jnp.concatenate(
        [h[:, 0:32] * ex[:, 0:1], h[:, 32:64] * ex[:, 1:2], ex, zp], axis=1)


def _bf16_dot(a, b):
    return jnp.dot(a.astype(jnp.bfloat16), b.astype(jnp.bfloat16),
                   preferred_element_type=jnp.float32)


def _pre_body(x_ref, w_ref, a_ref, hx_ref, ad_ref, sf_ref):
    h = _bf16_dot(x_ref[...], w_ref[...])
    asd = jnp.dot(h, a_ref[...], preferred_element_type=jnp.float32, precision=lax.Precision.HIGHEST)
    _pack_outputs(h, asd, hx_ref, ad_ref, sf_ref)


def _combine(acc_ref, sf_ref, b_ref):
    t = acc_ref[0] + acc_ref[1] + sf_ref[...]
    den0 = t[:, 64:65] + 1e-16
    den1 = t[:, 65:66] + 1e-16
    o = jnp.concatenate([t[:, 0:32] / den0, t[:, 32:64] / den1], axis=1) + b_ref[...]
    return jnp.where(o > 0, o, jnp.exp(jnp.minimum(o, 0.0)) - 1.0)


def _mid_body(acc_ref, sf_ref, b_ref, w_ref, a_ref, hx_ref, ad_ref, sf_out_ref):
    o = _combine(acc_ref, sf_ref, b_ref)
    h = _bf16_dot(o, w_ref[...])
    asd = jnp.dot(h, a_ref[...], preferred_element_type=jnp.float32, precision=lax.Precision.HIGHEST)
    _pack_outputs(h, asd, hx_ref, ad_ref, sf_out_ref)


def _layer_norm_rows(x, g, b):
    mu = jnp.mean(x, axis=-1, keepdims=True)
    var = jnp.mean((x - mu) ** 2, axis=-1, keepdims=True)
    return (x - mu) * lax.rsqrt(var + 1e-5) * g + b


def _post_body(acc_ref, sf_ref, b_ref, g_ref, gb2_ref, gw_ref, gb_ref,
               batch_ref, out_ref):
    i = pl.program_id(0)
    o = _combine(acc_ref, sf_ref, b_ref)
    hn = _layer_norm_rows(o, g_ref[...], gb2_ref[...])
    gate = _bf16_dot(hn, gw_ref[...]) + gb_ref[...]
    e = jnp.exp(gate)
    oh = (batch_ref[...] == lax.broadcasted_iota(jnp.int32, (o.shape[0], _G), 1))
    oh = oh.astype(jnp.float32)
    ex = jnp.concatenate([e * hn, e,
                          jnp.zeros((o.shape[0], _PK - 65), jnp.float32)],
                         axis=1)
    contrib = lax.dot_general(oh, ex, (((0,), (0,)), ((), ())),
                              preferred_element_type=jnp.float32, precision=lax.Precision.HIGHEST)

    @pl.when(i == 0)
    def _():
        out_ref[...] = jnp.zeros_like(out_ref)

    out_ref[...] += contrib


def _head_body(pa_ref, pc_ref, pp_ref, fng_ref, fnb_ref, fw_ref, fb_ref,
               flg_ref, flb_ref, c1w_ref, c1b_ref, c2w_ref, c2b_ref, out_ref):
    def unpool(p):
        return p[:, 0:64] / (p[:, 64:65] + 1e-16)

    comb = jnp.concatenate(
        [unpool(pa_ref[...]), unpool(pc_ref[...]), unpool(pp_ref[...])], axis=1)
    comb = _layer_norm_rows(comb, fng_ref[...], fnb_ref[...])
    fused = _bf16_dot(comb, fw_ref[...]) + fb_ref[...]
    fused = jnp.maximum(_layer_norm_rows(fused, flg_ref[...], flb_ref[...]), 0.0)
    h = jnp.maximum(_bf16_dot(fused, c1w_ref[...]) + c1b_ref[...], 0.0)
    out_ref[...] = _bf16_dot(h, c2w_ref[...]) + c2b_ref[...]


def _tc_pre(x, w, aext):
    n, f = x.shape
    return pl.pallas_call(
        _pre_body,
        grid=(n // _BLK,),
        in_specs=[
            pl.BlockSpec((_BLK, f), lambda i: (i, 0)),
            pl.BlockSpec((f, 64), lambda i: (0, 0)),
            pl.BlockSpec((64, 4), lambda i: (0, 0)),
        ],
        out_specs=[
            pl.BlockSpec((_BLK, _PK), lambda i: (i, 0)),
            pl.BlockSpec((_BLK, _ADW), lambda i: (i, 0)),
            pl.BlockSpec((_BLK, _PK), lambda i: (i, 0)),
        ],
        out_shape=[
            jax.ShapeDtypeStruct((n, _PK), jnp.float32),
            jax.ShapeDtypeStruct((n, _ADW), jnp.float32),
            jax.ShapeDtypeStruct((n, _PK), jnp.float32),
        ],
    )(x, w, aext)


def _tc_mid(acc, sf, bprev, w, aext):
    n = sf.shape[0]
    return pl.pallas_call(
        _mid_body,
        grid=(n // _BLK,),
        in_specs=[
            pl.BlockSpec((2, _BLK, _PK), lambda i: (0, i, 0)),
            pl.BlockSpec((_BLK, _PK), lambda i: (i, 0)),
            pl.BlockSpec((1, 64), lambda i: (0, 0)),
            pl.BlockSpec((64, 64), lambda i: (0, 0)),
            pl.BlockSpec((64, 4), lambda i: (0, 0)),
        ],
        out_specs=[
            pl.BlockSpec((_BLK, _PK), lambda i: (i, 0)),
            pl.BlockSpec((_BLK, _ADW), lambda i: (i, 0)),
            pl.BlockSpec((_BLK, _PK), lambda i: (i, 0)),
        ],
        out_shape=[
            jax.ShapeDtypeStruct((n, _PK), jnp.float32),
            jax.ShapeDtypeStruct((n, _ADW), jnp.float32),
            jax.ShapeDtypeStruct((n, _PK), jnp.float32),
        ],
    )(acc, sf, bprev, w, aext)


def _tc_post(acc, sf, bprev, ln_g, ln_b, gate_w, gate_b, batch2):
    n = sf.shape[0]
    return pl.pallas_call(
        _post_body,
        grid=(n // _BLK,),
        in_specs=[
            pl.BlockSpec((2, _BLK, _PK), lambda i: (0, i, 0)),
            pl.BlockSpec((_BLK, _PK), lambda i: (i, 0)),
            pl.BlockSpec((1, 64), lambda i: (0, 0)),
            pl.BlockSpec((1, 64), lambda i: (0, 0)),
            pl.BlockSpec((1, 64), lambda i: (0, 0)),
            pl.BlockSpec((64, 1), lambda i: (0, 0)),
            pl.BlockSpec((1, 1), lambda i: (0, 0)),
            pl.BlockSpec((_BLK, 1), lambda i: (i, 0)),
        ],
        out_specs=pl.BlockSpec((_G, _PK), lambda i: (0, 0)),
        out_shape=jax.ShapeDtypeStruct((_G, _PK), jnp.float32),
    )(acc, sf, bprev, ln_g, ln_b, gate_w, gate_b, batch2)


def _tc_head(pa, pc, pp, p):
    return pl.pallas_call(
        _head_body,
        out_shape=jax.ShapeDtypeStruct((_G, 1), jnp.float32),
    )(pa, pc, pp,
      p["fn_g"].reshape(1, -1), p["fn_b"].reshape(1, -1),
      p["fus_w"], p["fus_b"].reshape(1, -1),
      p["fln_g"].reshape(1, -1), p["fln_b"].reshape(1, -1),
      p["c1_w"], p["c1_b"].reshape(1, -1),
      p["c2_w"], p["c2_b"].reshape(1, -1))


def _aext_of(p):
    def blockdiag(a):
        m = jnp.zeros((2, 32, 2), jnp.float32)
        m = m.at[0, :, 0].set(a[0]).at[1, :, 1].set(a[1])
        return m.reshape(64, 2)

    return jnp.concatenate(
        [blockdiag(p["att_src"][0]), blockdiag(p["att_dst"][0])], axis=1)


def kernel(x, edge_index_ast, edge_index_cfg, edge_index_pdg, batch, params):
    batch2 = batch.astype(jnp.int32).reshape(_N, 1)

    def view(ei, p, tok):
        src = ei[0].astype(jnp.int32) + tok
        dst = ei[1].astype(jnp.int32)

        def run_edges(hx, ad):
            return _sc_edge(hx, jnp.ravel(ad[:, 2]), jnp.ravel(ad[:, 3]),
                            jnp.ravel(ad[:, 0]), jnp.ravel(ad[:, 1]),
                            src, dst)

        hx, ad, sf = _tc_pre(x, p["g1"]["W"], _aext_of(p["g1"]))
        acc = run_edges(hx, ad)
        hx, ad, sf = _tc_mid(acc, sf, p["g1"]["b"].reshape(1, 64),
                             p["g2"]["W"], _aext_of(p["g2"]))
        acc = run_edges(hx, ad)
        hx, ad, sf = _tc_mid(acc, sf, p["g2"]["b"].reshape(1, 64),
                             p["g3"]["W"], _aext_of(p["g3"]))
        acc = run_edges(hx, ad)
        return _tc_post(acc, sf, p["g3"]["b"].reshape(1, 64),
                        p["ln_g"].reshape(1, 64), p["ln_b"].reshape(1, 64),
                        p["gate_w"], p["gate_b"].reshape(1, 1), batch2)

    pa = view(edge_index_ast, params["ast"], jnp.zeros((), jnp.int32))
    pc = view(edge_index_cfg, params["cfg"], (pa[0, 0] * 0.0).astype(jnp.int32))
    pp = view(edge_index_pdg, params["pdg"], (pc[0, 0] * 0.0).astype(jnp.int32))
    return _tc_head(pa, pc, pp, params).reshape(-1)
